# Initial kernel scaffold; baseline (speedup 1.0000x reference)
import jax
import jax.numpy as jnp
from jax.experimental import pallas as pl


def kernel(xyz):
    raise NotImplementedError("write your pallas kernel here")


# speedup vs baseline: 8.5769x; 8.5769x over previous
import functools

import jax
import jax.numpy as jnp
from jax import lax
from jax.experimental import pallas as pl
from jax.experimental.pallas import tpu as pltpu
from jax.experimental.pallas import tpu_sc as plsc

B = 16
N = 8192
G = 1024
K = 32


def _fps_body(x_ref, y_ref, z_ref, idx_ref, cx_ref, cy_ref, cz_ref, m_ref,
              dist_ref, far_ref):
    i = pl.program_id(0)

    @pl.when(i == 0)
    def _init():
        dist_ref[...] = jnp.full((B, N), 1e10, dtype=jnp.float32)
        far_ref[...] = jnp.zeros((B, 1), dtype=jnp.int32)

    far = far_ref[...]
    idx_ref[...] = far[None]

    lane = jax.lax.broadcasted_iota(jnp.int32, (B, N), 1)
    sel = lane == far
    x = x_ref[...]
    y = y_ref[...]
    z = z_ref[...]
    cx = jnp.max(jnp.where(sel, x, -jnp.inf), axis=1, keepdims=True)
    cy = jnp.max(jnp.where(sel, y, -jnp.inf), axis=1, keepdims=True)
    cz = jnp.max(jnp.where(sel, z, -jnp.inf), axis=1, keepdims=True)
    cx_ref[...] = cx[None]
    cy_ref[...] = cy[None]
    cz_ref[...] = cz[None]

    dx = x - cx
    dy = y - cy
    dz = z - cz
    d = (dx * dx + dz * dz) + dy * dy
    dist = dist_ref[...]
    dist = jnp.where(d < dist, d, dist)
    dist_ref[...] = dist

    m = jnp.max(dist, axis=1, keepdims=True)
    first = jnp.min(jnp.where(dist == m, lane, N), axis=1, keepdims=True)
    far_ref[...] = first
    m_ref[...] = m[None]


def _fps(x, y, z):
    out = pl.pallas_call(
        _fps_body,
        grid=(G,),
        in_specs=[pl.BlockSpec((B, N), lambda i: (0, 0))] * 3,
        out_specs=[
            pl.BlockSpec((1, B, 1), lambda i: (i, 0, 0)),
            pl.BlockSpec((1, B, 1), lambda i: (i, 0, 0)),
            pl.BlockSpec((1, B, 1), lambda i: (i, 0, 0)),
            pl.BlockSpec((1, B, 1), lambda i: (i, 0, 0)),
            pl.BlockSpec((1, B, 1), lambda i: (i, 0, 0)),
        ],
        out_shape=[
            jax.ShapeDtypeStruct((G, B, 1), jnp.int32),
            jax.ShapeDtypeStruct((G, B, 1), jnp.float32),
            jax.ShapeDtypeStruct((G, B, 1), jnp.float32),
            jax.ShapeDtypeStruct((G, B, 1), jnp.float32),
            jax.ShapeDtypeStruct((G, B, 1), jnp.float32),
        ],
        scratch_shapes=[
            pltpu.VMEM((B, N), jnp.float32),
            pltpu.VMEM((B, 1), jnp.int32),
        ],
    )(x, y, z)
    return out


TG = 128


def _knn_body(cx_ref, cy_ref, cz_ref, x_ref, y_ref, z_ref, idx_ref):
    cx = cx_ref[...]
    cy = cy_ref[...]
    cz = cz_ref[...]
    x = x_ref[...].reshape(1, N)
    y = y_ref[...].reshape(1, N)
    z = z_ref[...].reshape(1, N)

    sn = (cx * cx + cz * cz) + cy * cy
    sx = (x * x + z * z) + y * y
    xb = x.astype(jnp.bfloat16).astype(jnp.float32)
    yb = y.astype(jnp.bfloat16).astype(jnp.float32)
    zb = z.astype(jnp.bfloat16).astype(jnp.float32)
    cxb = cx.astype(jnp.bfloat16).astype(jnp.float32)
    cyb = cy.astype(jnp.bfloat16).astype(jnp.float32)
    czb = cz.astype(jnp.bfloat16).astype(jnp.float32)
    t = cxb * xb
    t = t + cyb * yb
    t = t + czb * zb
    d2 = (sn + sx) - 2.0 * t
    dist = jnp.sqrt(jnp.maximum(d2, 0.0))

    lane = jax.lax.broadcasted_iota(jnp.int32, (TG, N), 1)
    kiota = jax.lax.broadcasted_iota(jnp.int32, (TG, K), 1)
    acc = jnp.zeros((TG, K), dtype=jnp.int32)
    for j in range(K):
        m = jnp.min(dist, axis=1, keepdims=True)
        idx = jnp.min(jnp.where(dist == m, lane, N), axis=1, keepdims=True)
        acc = jnp.where(kiota == j, idx, acc)
        dist = jnp.where(lane == idx, jnp.inf, dist)
    idx_ref[...] = acc[None]


def _knn(cxf, cyf, czf, x3, y3, z3):
    ntg = G // TG
    return pl.pallas_call(
        _knn_body,
        grid=(B, ntg),
        in_specs=[
            pl.BlockSpec((TG, 1), lambda b, g: (b * ntg + g, 0)),
            pl.BlockSpec((TG, 1), lambda b, g: (b * ntg + g, 0)),
            pl.BlockSpec((TG, 1), lambda b, g: (b * ntg + g, 0)),
            pl.BlockSpec((1, 1, N), lambda b, g: (b, 0, 0)),
            pl.BlockSpec((1, 1, N), lambda b, g: (b, 0, 0)),
            pl.BlockSpec((1, 1, N), lambda b, g: (b, 0, 0)),
        ],
        out_specs=pl.BlockSpec((1, TG, K), lambda b, g: (b, g, 0)),
        out_shape=jax.ShapeDtypeStruct((B, G, K), jnp.int32),
    )(cxf, cyf, czf, x3, y3, z3)


_NC, _NS = 2, 16
_NW = _NC * _NS
_ELW = G * K * 3 // 2
_ROWS = _ELW // 128


@functools.partial(
    pl.kernel,
    mesh=plsc.VectorSubcoreMesh(core_axis_name="c", subcore_axis_name="s"),
    out_type=jax.ShapeDtypeStruct((_NW, _ELW), jnp.float32),
    scratch_types=[
        pltpu.VMEM((_ROWS, 128), jnp.int32),
        pltpu.VMEM((_ELW,), jnp.float32),
        pltpu.SemaphoreType.DMA,
    ],
)
def _sc_gather_kernel(tab_hbm, gi_hbm, out_hbm, gi_v, out_v, sem):
    wid = lax.axis_index("s") * _NC + lax.axis_index("c")
    pltpu.sync_copy(gi_hbm.at[wid], gi_v)

    def body(j, carry):
        pltpu.async_copy(tab_hbm.at[gi_v.at[j]],
                         out_v.at[pl.ds(j * 128, 128)], sem).wait()
        return carry

    lax.fori_loop(0, _ROWS, body, 0)
    pltpu.sync_copy(out_v, out_hbm.at[wid])


def kernel(xyz):
    x = xyz[:, :, 0]
    y = xyz[:, :, 1]
    z = xyz[:, :, 2]
    idx3, cx3, cy3, cz3, _m3 = _fps(x, y, z)
    center = jnp.stack([cx3[:, :, 0].T, cy3[:, :, 0].T, cz3[:, :, 0].T],
                       axis=-1)

    cxf = cx3[:, :, 0].T.reshape(B * G, 1)
    cyf = cy3[:, :, 0].T.reshape(B * G, 1)
    czf = cz3[:, :, 0].T.reshape(B * G, 1)
    group_idx = _knn(cxf, cyf, czf, x[:, None, :], y[:, None, :],
                     z[:, None, :])

    tab = xyz.reshape(B * N * 3)
    base = (jnp.arange(B, dtype=jnp.int32) * N)[:, None, None]
    rowidx = (group_idx + base).reshape(B, G * K)
    eidx = rowidx[:, :, None] * 3 + jnp.arange(3, dtype=jnp.int32)
    gidx = eidx.reshape(_NW, _ROWS, 128)
    gathered = _sc_gather_kernel(tab, gidx).reshape(B, G, K, 3)
    neighborhood = gathered - center[:, :, None, :]
    return (neighborhood, center, group_idx)


# speedup vs baseline: 8.7973x; 1.0257x over previous
import functools

import jax
import jax.numpy as jnp
from jax import lax
from jax.experimental import pallas as pl
from jax.experimental.pallas import tpu as pltpu
from jax.experimental.pallas import tpu_sc as plsc

B = 16
N = 8192
G = 1024
K = 32


def _fps_body(x_ref, y_ref, z_ref, idx_ref, cx_ref, cy_ref, cz_ref, m_ref,
              dist_ref, far_ref):
    i = pl.program_id(0)

    @pl.when(i == 0)
    def _init():
        dist_ref[...] = jnp.full((B, N), 1e10, dtype=jnp.float32)
        far_ref[...] = jnp.zeros((B, 1), dtype=jnp.int32)

    far = far_ref[...]
    idx_ref[...] = far[None]

    lane = jax.lax.broadcasted_iota(jnp.int32, (B, N), 1)
    sel = lane == far
    x = x_ref[...]
    y = y_ref[...]
    z = z_ref[...]
    cx = jnp.max(jnp.where(sel, x, -jnp.inf), axis=1, keepdims=True)
    cy = jnp.max(jnp.where(sel, y, -jnp.inf), axis=1, keepdims=True)
    cz = jnp.max(jnp.where(sel, z, -jnp.inf), axis=1, keepdims=True)
    cx_ref[...] = cx[None]
    cy_ref[...] = cy[None]
    cz_ref[...] = cz[None]

    dx = x - cx
    dy = y - cy
    dz = z - cz
    d = (dx * dx + dz * dz) + dy * dy
    dist = dist_ref[...]
    dist = jnp.where(d < dist, d, dist)
    dist_ref[...] = dist

    m = jnp.max(dist, axis=1, keepdims=True)
    first = jnp.min(jnp.where(dist == m, lane, N), axis=1, keepdims=True)
    far_ref[...] = first
    m_ref[...] = m[None]


def _fps(x, y, z):
    out = pl.pallas_call(
        _fps_body,
        grid=(G,),
        in_specs=[pl.BlockSpec((B, N), lambda i: (0, 0))] * 3,
        out_specs=[
            pl.BlockSpec((1, B, 1), lambda i: (i, 0, 0)),
            pl.BlockSpec((1, B, 1), lambda i: (i, 0, 0)),
            pl.BlockSpec((1, B, 1), lambda i: (i, 0, 0)),
            pl.BlockSpec((1, B, 1), lambda i: (i, 0, 0)),
            pl.BlockSpec((1, B, 1), lambda i: (i, 0, 0)),
        ],
        out_shape=[
            jax.ShapeDtypeStruct((G, B, 1), jnp.int32),
            jax.ShapeDtypeStruct((G, B, 1), jnp.float32),
            jax.ShapeDtypeStruct((G, B, 1), jnp.float32),
            jax.ShapeDtypeStruct((G, B, 1), jnp.float32),
            jax.ShapeDtypeStruct((G, B, 1), jnp.float32),
        ],
        scratch_shapes=[
            pltpu.VMEM((B, N), jnp.float32),
            pltpu.VMEM((B, 1), jnp.int32),
        ],
    )(x, y, z)
    return out


TG = 128


def _knn_body(cx_ref, cy_ref, cz_ref, x_ref, y_ref, z_ref, idx_ref):
    cx = cx_ref[...]
    cy = cy_ref[...]
    cz = cz_ref[...]
    x = x_ref[...].reshape(1, N)
    y = y_ref[...].reshape(1, N)
    z = z_ref[...].reshape(1, N)

    sn = (cx * cx + cz * cz) + cy * cy
    sx = (x * x + z * z) + y * y
    xb = x.astype(jnp.bfloat16).astype(jnp.float32)
    yb = y.astype(jnp.bfloat16).astype(jnp.float32)
    zb = z.astype(jnp.bfloat16).astype(jnp.float32)
    cxb = cx.astype(jnp.bfloat16).astype(jnp.float32)
    cyb = cy.astype(jnp.bfloat16).astype(jnp.float32)
    czb = cz.astype(jnp.bfloat16).astype(jnp.float32)
    t = cxb * xb
    t = t + cyb * yb
    t = t + czb * zb
    d2 = (sn + sx) - 2.0 * t
    dist = jnp.sqrt(jnp.maximum(d2, 0.0))

    lane = jax.lax.broadcasted_iota(jnp.int32, (TG, N), 1)
    kiota = jax.lax.broadcasted_iota(jnp.int32, (TG, K), 1)
    acc = jnp.zeros((TG, K), dtype=jnp.int32)
    for j in range(K):
        m = jnp.min(dist, axis=1, keepdims=True)
        idx = jnp.min(jnp.where(dist == m, lane, N), axis=1, keepdims=True)
        acc = jnp.where(kiota == j, idx, acc)
        dist = jnp.where(lane == idx, jnp.inf, dist)
    idx_ref[...] = acc[None]


def _knn(cxf, cyf, czf, x3, y3, z3):
    ntg = G // TG
    return pl.pallas_call(
        _knn_body,
        grid=(B, ntg),
        in_specs=[
            pl.BlockSpec((TG, 1), lambda b, g: (b * ntg + g, 0)),
            pl.BlockSpec((TG, 1), lambda b, g: (b * ntg + g, 0)),
            pl.BlockSpec((TG, 1), lambda b, g: (b * ntg + g, 0)),
            pl.BlockSpec((1, 1, N), lambda b, g: (b, 0, 0)),
            pl.BlockSpec((1, 1, N), lambda b, g: (b, 0, 0)),
            pl.BlockSpec((1, 1, N), lambda b, g: (b, 0, 0)),
        ],
        out_specs=pl.BlockSpec((1, TG, K), lambda b, g: (b, g, 0)),
        out_shape=jax.ShapeDtypeStruct((B, G, K), jnp.int32),
    )(cxf, cyf, czf, x3, y3, z3)


_NC, _NS = 2, 16
_NW = _NC * _NS
_ELW = G * K * 3 // 2
_ROWS = _ELW // 128


@functools.partial(
    pl.kernel,
    mesh=plsc.VectorSubcoreMesh(core_axis_name="c", subcore_axis_name="s"),
    out_type=jax.ShapeDtypeStruct((_NW, _ELW), jnp.float32),
    scratch_types=[
        pltpu.VMEM((_ROWS, 128), jnp.int32),
        pltpu.VMEM((_ELW,), jnp.float32),
        pltpu.SemaphoreType.DMA,
    ],
)
def _sc_gather_kernel(tab_hbm, gi_hbm, out_hbm, gi_v, out_v, sem):
    wid = lax.axis_index("s") * _NC + lax.axis_index("c")
    pltpu.sync_copy(gi_hbm.at[wid], gi_v)

    def body(s, carry):
        for t in range(24):
            j = s * 24 + t
            pltpu.make_async_copy(tab_hbm.at[gi_v.at[j]],
                                  out_v.at[pl.ds(j * 128, 128)], sem).start()
        for t in range(24):
            j = s * 24 + t
            pltpu.make_async_copy(tab_hbm.at[gi_v.at[j]],
                                  out_v.at[pl.ds(j * 128, 128)], sem).wait()
        return carry

    lax.fori_loop(0, _ROWS // 24, body, 0)
    pltpu.sync_copy(out_v, out_hbm.at[wid])


def kernel(xyz):
    x = xyz[:, :, 0]
    y = xyz[:, :, 1]
    z = xyz[:, :, 2]
    idx3, cx3, cy3, cz3, _m3 = _fps(x, y, z)
    center = jnp.stack([cx3[:, :, 0].T, cy3[:, :, 0].T, cz3[:, :, 0].T],
                       axis=-1)

    cxf = cx3[:, :, 0].T.reshape(B * G, 1)
    cyf = cy3[:, :, 0].T.reshape(B * G, 1)
    czf = cz3[:, :, 0].T.reshape(B * G, 1)
    group_idx = _knn(cxf, cyf, czf, x[:, None, :], y[:, None, :],
                     z[:, None, :])

    tab = xyz.reshape(B * N * 3)
    base = (jnp.arange(B, dtype=jnp.int32) * N)[:, None, None]
    rowidx = (group_idx + base).reshape(B, G * K)
    eidx = rowidx[:, :, None] * 3 + jnp.arange(3, dtype=jnp.int32)
    gidx = eidx.reshape(_NW, _ROWS, 128)
    gathered = _sc_gather_kernel(tab, gidx).reshape(B, G, K, 3)
    neighborhood = gathered - center[:, :, None, :]
    return (neighborhood, center, group_idx)
